# Initial kernel scaffold; baseline (speedup 1.0000x reference)
import jax
import jax.numpy as jnp
from jax.experimental import pallas as pl


def kernel(x, A, W, temperature):
    raise NotImplementedError("write your pallas kernel here")


# speedup vs baseline: 21.0203x; 21.0203x over previous
import functools

import jax
import jax.numpy as jnp
from jax.experimental import pallas as pl
from jax.experimental.pallas import tpu as pltpu

_N = 4096
_D = 256
_K = 10
_BR = 512


def _proj_kernel(x_ref, w_ref, xp_ref):
    xp_ref[...] = jax.lax.dot_general(
        x_ref[...], w_ref[...], (((1,), (0,)), ((), ())),
        preferred_element_type=jnp.float32)


def _dist_topk_kernel(t_ref, xpb_ref, xp_ref, vals_ref, idx_ref):
    xb = xpb_ref[...]
    xf = xp_ref[...]
    t = t_ref[0, 0]
    g = jax.lax.dot_general(
        xb, xf, (((1,), (1,)), ((), ())),
        preferred_element_type=jnp.float32)
    sqb = jnp.sum(xb * xb, axis=1)[:, None]
    sqf = jnp.sum(xf * xf, axis=1)[None, :]
    lq = (sqb + sqf - 2.0 * g) * t
    iota = jax.lax.broadcasted_iota(jnp.int32, lq.shape, 1)
    big = jnp.float32(jnp.inf)
    vals, idxs = [], []
    for _ in range(_K):
        m = jnp.min(lq, axis=1)
        im = jnp.min(jnp.where(lq == m[:, None], iota, _N), axis=1)
        vals.append(-m)
        idxs.append(im)
        lq = jnp.where(iota == im[:, None], big, lq)
    vals_ref[...] = jnp.stack(vals, axis=1)
    idx_ref[...] = jnp.stack(idxs, axis=1)


@functools.partial(jax.jit, static_argnames=())
def kernel(x, A, W, temperature):
    del A
    n, d = x.shape
    t = jnp.exp(jnp.clip(temperature, -5.0, 5.0)).reshape(1, 1)

    xp = pl.pallas_call(
        _proj_kernel,
        out_shape=jax.ShapeDtypeStruct((n, d), jnp.float32),
    )(x, W)

    grid = (n // _BR,)
    vals, idx = pl.pallas_call(
        _dist_topk_kernel,
        grid=grid,
        in_specs=[
            pl.BlockSpec((1, 1), lambda i: (0, 0), memory_space=pltpu.SMEM),
            pl.BlockSpec((_BR, d), lambda i: (i, 0)),
            pl.BlockSpec((n, d), lambda i: (0, 0)),
        ],
        out_specs=[
            pl.BlockSpec((_BR, _K), lambda i: (i, 0)),
            pl.BlockSpec((_BR, _K), lambda i: (i, 0)),
        ],
        out_shape=[
            jax.ShapeDtypeStruct((n, _K), jnp.float32),
            jax.ShapeDtypeStruct((n, _K), jnp.int32),
        ],
    )(t, xp, xp)

    logprobs = vals[None]
    rows = jnp.repeat(jnp.arange(n, dtype=jnp.int32), _K)
    edges_hat = jnp.stack([idx.reshape(-1), rows], axis=0)
    return (xp[None], edges_hat, logprobs)


# speedup vs baseline: 37.7606x; 1.7964x over previous
import functools

import jax
import jax.numpy as jnp
from jax.experimental import pallas as pl
from jax.experimental.pallas import tpu as pltpu

_N = 4096
_D = 256
_K = 10
_BR = 512
_NSEG = 32
_T = 4


def _proj_kernel(x_ref, w_ref, xp_ref):
    xp_ref[...] = jax.lax.dot_general(
        x_ref[...], w_ref[...], (((1,), (0,)), ((), ())),
        preferred_element_type=jnp.float32)


def _dist_topk_kernel(t_ref, xpb_ref, xp_ref, vals_ref, idx_ref):
    xb = xpb_ref[...]
    xf = xp_ref[...]
    t = t_ref[0, 0]
    g = jax.lax.dot_general(
        xb, xf, (((1,), (1,)), ((), ())),
        preferred_element_type=jnp.float32)
    sqb = jnp.sum(xb * xb, axis=1)[:, None]
    sqf = jnp.sum(xf * xf, axis=1)[None, :]
    d2 = (sqb + sqf) - 2.0 * g
    inf = jnp.float32(jnp.inf)

    masked = [d2[:, j * 128:(j + 1) * 128] for j in range(_NSEG)]
    vs, js = [], []
    for r in range(_T):
        cur = masked[0]
        icur = jnp.zeros(cur.shape, jnp.int32)
        for j in range(1, _NSEG):
            ltm = masked[j] < cur
            cur = jnp.where(ltm, masked[j], cur)
            icur = jnp.where(ltm, j, icur)
        vs.append(cur)
        js.append(icur)
        if r + 1 < _T:
            masked = [jnp.where(mj == cur, inf, mj) for mj in masked]

    lane = jax.lax.broadcasted_iota(jnp.int32, vs[0].shape, 1)
    cur, c2, c3, c4 = vs
    icur, ic2, ic3, _ = js
    outv, outi = [], []
    for _ in range(_K):
        m = jnp.min(cur, axis=1)
        hit = cur == m[:, None]
        o = jnp.min(jnp.where(hit, lane, _N), axis=1)
        win = lane == o[:, None]
        j32 = jnp.min(jnp.where(win, icur, _NSEG), axis=1)
        outv.append(m)
        outi.append(j32 * 128 + o)
        cur = jnp.where(win, c2, cur)
        c2 = jnp.where(win, c3, c2)
        c3 = jnp.where(win, c4, c3)
        c4 = jnp.where(win, inf, c4)
        icur = jnp.where(win, ic2, icur)
        ic2 = jnp.where(win, ic3, ic2)
    vals_ref[...] = jnp.stack(outv, axis=1) * (-t)
    idx_ref[...] = jnp.stack(outi, axis=1)


@functools.partial(jax.jit, static_argnames=())
def kernel(x, A, W, temperature):
    del A
    n, d = x.shape
    t = jnp.exp(jnp.clip(temperature, -5.0, 5.0)).reshape(1, 1)

    xp = pl.pallas_call(
        _proj_kernel,
        out_shape=jax.ShapeDtypeStruct((n, d), jnp.float32),
    )(x, W)

    grid = (n // _BR,)
    vals, idx = pl.pallas_call(
        _dist_topk_kernel,
        grid=grid,
        in_specs=[
            pl.BlockSpec((1, 1), lambda i: (0, 0), memory_space=pltpu.SMEM),
            pl.BlockSpec((_BR, d), lambda i: (i, 0)),
            pl.BlockSpec((n, d), lambda i: (0, 0)),
        ],
        out_specs=[
            pl.BlockSpec((_BR, _K), lambda i: (i, 0)),
            pl.BlockSpec((_BR, _K), lambda i: (i, 0)),
        ],
        out_shape=[
            jax.ShapeDtypeStruct((n, _K), jnp.float32),
            jax.ShapeDtypeStruct((n, _K), jnp.int32),
        ],
    )(t, xp, xp)

    logprobs = vals[None]
    rows = jnp.repeat(jnp.arange(n, dtype=jnp.int32), _K)
    edges_hat = jnp.stack([idx.reshape(-1), rows], axis=0)
    return (xp[None], edges_hat, logprobs)
